# baseline (speedup 1.0000x reference)
import functools

import jax
import jax.numpy as jnp
from jax import lax
from jax.experimental import pallas as pl
from jax.experimental.pallas import tpu as pltpu
from jax.experimental.pallas import tpu_sc as plsc

VOCAB = 1000000
EMB = 64
N_TOK = 1024 * 200

_NC = 2
_NS = 16
_NW = _NC * _NS

_PER_W = N_TOK // _NW
_CHUNK = 800
_NCHUNK = _PER_W // _CHUNK


def _emb_body(src_hbm, table_hbm, out_hbm, idx_v, buf0, buf1, gsem0, gsem1,
              wsem0, wsem1):
    wid = lax.axis_index("s") * _NC + lax.axis_index("c")
    base = wid * _PER_W
    pltpu.sync_copy(src_hbm.at[pl.ds(base, _PER_W)], idx_v)

    bufs = (buf0, buf1)
    gsems = (gsem0, gsem1)
    wsems = (wsem0, wsem1)

    def gather(c):
        b = c % 2
        return pltpu.async_copy(
            table_hbm.at[idx_v.at[pl.ds(c * _CHUNK, _CHUNK)]], bufs[b],
            gsems[b])

    def write(c):
        b = c % 2
        return pltpu.async_copy(
            bufs[b], out_hbm.at[pl.ds(base + c * _CHUNK, _CHUNK)], wsems[b])

    g = [None] * _NCHUNK
    w = [None] * _NCHUNK
    g[0] = gather(0)
    g[1] = gather(1)
    for c in range(_NCHUNK):
        g[c].wait()
        w[c] = write(c)
        if c + 2 < _NCHUNK:
            w[c].wait()
            g[c + 2] = gather(c + 2)
    w[_NCHUNK - 2].wait()
    w[_NCHUNK - 1].wait()


@jax.jit
def _embedding_lookup(src_flat, table):
    mesh = plsc.VectorSubcoreMesh(core_axis_name="c", subcore_axis_name="s")
    fn = functools.partial(
        pl.kernel,
        mesh=mesh,
        out_type=jax.ShapeDtypeStruct((N_TOK, EMB), jnp.float32),
        scratch_types=[
            pltpu.VMEM((_PER_W,), jnp.int32),
            pltpu.VMEM((_CHUNK, EMB), jnp.float32),
            pltpu.VMEM((_CHUNK, EMB), jnp.float32),
            pltpu.SemaphoreType.DMA,
            pltpu.SemaphoreType.DMA,
            pltpu.SemaphoreType.DMA,
            pltpu.SemaphoreType.DMA,
        ],
        compiler_params=pltpu.CompilerParams(use_tc_tiling_on_sc=False),
    )(_emb_body)
    return fn(src_flat, table)


def kernel(src, seg, table):
    del seg
    src_flat = src.reshape(-1).astype(jnp.int32)
    out = _embedding_lookup(src_flat, table)
    return out.reshape(src.shape[0], src.shape[1], EMB)
